# baseline (speedup 1.0000x reference)
import functools

import jax
import jax.numpy as jnp
from jax import lax
from jax.experimental import pallas as pl
from jax.experimental.pallas import tpu as pltpu
from jax.experimental.pallas import tpu_sc as plsc

B, L, H = 4, 8192, 1024
N = B * L
NC, NS = 2, 16
NW = NC * NS
CH = 8
NBUF = 4
DEPTH = 3
VPR = H // 16


def _sc_lookup_add(xf, idx, emb, n_rows):
    rows_per_w = n_rows // NW
    nchunk = rows_per_w // CH
    mesh = plsc.VectorSubcoreMesh(core_axis_name="c", subcore_axis_name="s")

    @functools.partial(
        pl.kernel,
        out_type=jax.ShapeDtypeStruct((n_rows, H), jnp.float32),
        mesh=mesh,
        scratch_types=[
            pltpu.VMEM((rows_per_w,), jnp.int32),
            pltpu.VMEM((NBUF, CH, H), jnp.float32),
            pltpu.VMEM((NBUF, CH, H), jnp.float32),
            pltpu.SemaphoreType.DMA((NBUF,)),
            pltpu.SemaphoreType.DMA((NBUF,)),
            pltpu.SemaphoreType.DMA((NBUF,)),
        ],
    )
    def k(x_hbm, idx_hbm, emb_hbm, out_hbm, idx_v, rows_v, x_v,
          gsem, xsem, osem):
        wid = lax.axis_index("s") * NC + lax.axis_index("c")
        base = wid * rows_per_w
        pltpu.sync_copy(idx_hbm.at[pl.ds(base, rows_per_w)], idx_v)

        def issue_inputs(cc, b):
            pltpu.async_copy(
                emb_hbm.at[idx_v.at[pl.ds(cc * CH, CH)]],
                rows_v.at[b], gsem.at[b])
            pltpu.async_copy(
                x_hbm.at[pl.ds(base + cc * CH, CH)], x_v.at[b], xsem.at[b])

        def drain_out(cc, b):
            pltpu.make_async_copy(
                rows_v.at[b], out_hbm.at[pl.ds(base + cc * CH, CH)],
                osem.at[b]).wait()

        for b in range(DEPTH):
            issue_inputs(b, b)

        @pl.loop(0, nchunk, step=NBUF)
        def super_body(c):
            for b in range(NBUF):
                cc = c + b
                nc = cc + DEPTH
                bn = (b + DEPTH) % NBUF

                @pl.when(nc < nchunk)
                def _():
                    @pl.when(cc >= NBUF - DEPTH)
                    def _():
                        drain_out(cc - (NBUF - DEPTH), bn)
                    issue_inputs(nc, bn)

                pltpu.make_async_copy(
                    emb_hbm.at[idx_v.at[pl.ds(cc * CH, CH)]],
                    rows_v.at[b], gsem.at[b]).wait()
                pltpu.make_async_copy(
                    x_hbm.at[pl.ds(base + cc * CH, CH)], x_v.at[b],
                    xsem.at[b]).wait()

                @pl.loop(0, CH)
                def row_body(r):
                    @pl.loop(0, VPR, unroll=8)
                    def vec_body(j):
                        v = x_v[b, r, pl.ds(j * 16, 16)]
                        plsc.addupdate(rows_v.at[b, r, pl.ds(j * 16, 16)], v)

                pltpu.async_copy(
                    rows_v.at[b], out_hbm.at[pl.ds(base + cc * CH, CH)],
                    osem.at[b])

        for b in range(NBUF):
            cc = nchunk - NBUF + b
            drain_out(cc, b)

    return k(xf, idx, emb)


def kernel(x, pos, emb):
    xf = x.reshape(N, H)
    idx = pos.reshape(N).astype(jnp.int32)
    out = _sc_lookup_add(xf, idx, emb, N)
    return out.reshape(B, L, H)
